# Initial kernel scaffold; baseline (speedup 1.0000x reference)
import jax
import jax.numpy as jnp
from jax.experimental import pallas as pl


def kernel(x, params, edge_index, batch):
    raise NotImplementedError("write your pallas kernel here")


# speedup vs baseline: 1.0032x; 1.0032x over previous
import functools

import jax
import jax.numpy as jnp
from jax.experimental import pallas as pl

N = 10000
E = 320000
NG = 64
H = 4
HC = 64
NL = 4
EPS = 1e-5


def _ln_body(x_ref, g_ref, b_ref, o_ref, *, do_gelu):
    x = x_ref[...]
    m = jnp.mean(x, axis=-1, keepdims=True)
    v = jnp.mean((x - m) ** 2, axis=-1, keepdims=True)
    y = (x - m) * jax.lax.rsqrt(v + EPS) * g_ref[...] + b_ref[...]
    if do_gelu:
        y = 0.5 * y * (1.0 + jax.lax.erf(y * (2.0 ** -0.5)))
    o_ref[...] = y


def _ln(x, g, b, do_gelu):
    n, d = x.shape
    blk = 1000 if n % 1000 == 0 else n
    grid = n // blk
    return pl.pallas_call(
        functools.partial(_ln_body, do_gelu=do_gelu),
        grid=(grid,),
        in_specs=[
            pl.BlockSpec((blk, d), lambda i: (i, 0)),
            pl.BlockSpec((d,), lambda i: (0,)),
            pl.BlockSpec((d,), lambda i: (0,)),
        ],
        out_specs=pl.BlockSpec((blk, d), lambda i: (i, 0)),
        out_shape=jax.ShapeDtypeStruct((n, d), x.dtype),
    )(x, g, b)


def _gat_conv(x, src, dst, W, a_s, a_d, bias):
    h = (x @ W).reshape(-1, H, HC)
    al_s = jnp.sum(h * a_s, axis=-1)
    al_d = jnp.sum(h * a_d, axis=-1)
    e = al_s[src] + al_d[dst]
    e = jnp.where(e > 0, e, 0.2 * e)
    mx = jax.ops.segment_max(e, dst, num_segments=N)
    ex = jnp.exp(e - mx[dst])
    den = jax.ops.segment_sum(ex, dst, num_segments=N)
    alpha = ex / (den[dst] + 1e-16)
    out = jax.ops.segment_sum(h[src] * alpha[:, :, None], dst, num_segments=N)
    return out.reshape(-1, H * HC) + bias


def kernel(x, params, edge_index, batch):
    p = params
    loop = jnp.arange(N)
    src = jnp.concatenate([edge_index[0], loop])
    dst = jnp.concatenate([edge_index[1], loop])

    h = _ln(x @ p['in_W'] + p['in_b'], p['in_g'], p['in_be'], True)
    for i in range(NL):
        idn = h @ p['sk_W%d' % i] + p['sk_b%d' % i]
        h = _gat_conv(h, src, dst, p['gat_W%d' % i], p['att_s%d' % i],
                      p['att_d%d' % i], p['gat_b%d' % i])
        h = _ln(h, p['ln_g%d' % i], p['ln_b%d' % i], True) + idn
    h = _ln(h @ p['pp_W1'] + p['pp_b1'], p['pp_g1'], p['pp_be1'], True)
    h = _ln(h @ p['pp_W2'] + p['pp_b2'], p['pp_g2'], p['pp_be2'], False)

    pw = jax.nn.softmax(p['pool_w'])
    ones = jnp.ones((h.shape[0], 1), jnp.float32)
    cnt = jax.ops.segment_sum(ones, batch, num_segments=NG)
    s = jax.ops.segment_sum(h, batch, num_segments=NG)
    x_mean = s / jnp.maximum(cnt, 1.0) * pw[0]
    mx = jax.ops.segment_max(h, batch, num_segments=NG)
    mx = jnp.where(jnp.isfinite(mx), mx, 0.0)
    x_max = mx * pw[1]
    x_add = s * pw[2]
    g = jnp.concatenate([x_mean, x_max, x_add], axis=1)
    g = _ln(g @ p['cl_W1'] + p['cl_b1'], p['cl_g1'], p['cl_be1'], True)
    g = _ln(g @ p['cl_W2'] + p['cl_b2'], p['cl_g2'], p['cl_be2'], True)
    return g @ p['cl_W3'] + p['cl_b3']
